# baseline (speedup 1.0000x reference)
import functools

import jax
import jax.numpy as jnp
from jax import lax
from jax.experimental import pallas as pl
from jax.experimental.pallas import tpu as pltpu
from jax.experimental.pallas import tpu_sc as plsc

B = 16384
D = 64
L = 16
NC = 2
NS = 16
NW = NC * NS
V = 1000000
SD = 64

PANELS = 61
WIN = 128
RNG = PANELS * WIN
NWIN = 244
RSZ = NWIN * WIN
TAIL0 = 31 * RSZ

HITCAP = 1024
RING = 8

_SENTINEL = 0x3FFFFFFF


def _scan_hits(idx_v, hits_v, r0, rsz):
    big = jnp.full((L,), _SENTINEL, jnp.int32)

    def prefill(k, _):
        hits_v[pl.ds(k * L, L)] = big
        return 0
    lax.fori_loop(0, HITCAP // L, prefill, 0)

    lanes = lax.iota(jnp.int32, L)

    def step(j, n):
        vec = idx_v[pl.ds(j * L, L)]
        rl = vec - r0
        mask = (rl >= 0) & (rl < rsz)
        key = rl * 16384 + (j * L + lanes)
        plsc.store_compressed(hits_v.at[pl.ds(n, L)], key, mask=mask)
        cnt = plsc.all_reduce_population_count(mask)[0]
        return jnp.minimum(n + cnt, HITCAP - L)

    return lax.fori_loop(0, B // L, step, 0)


def _extract_window(win_v, hits_v, nv, w, width, rowbuf_v, stage_hbm,
                    sem, cnt0):
    lanes = lax.iota(jnp.int32, L)

    def jstep(jv, cnt):
        kvec = hits_v[pl.ds(jv * L, L)]
        wmask = (kvec >> 21) == w

        def has(m):
            return plsc.all_reduce_population_count(m)[0] > 0

        def body(carry):
            m, c = carry
            lane = plsc.all_reduce_ffs(m)[0]
            keyv = jnp.take_along_axis(
                kvec, jnp.full((L,), 0, jnp.int32) + lane, axis=0,
                mode="promise_in_bounds")
            relv = (keyv >> 14) - w * WIN
            b = keyv[0] & 16383
            slot = c & (RING - 1)
            for g in range(D // L):
                vals = plsc.load_gather(win_v, [g * L + lanes, relv])
                rowbuf_v[slot, pl.ds(g * L, L)] = vals
            @pl.when(c >= RING)
            def _():
                pltpu.make_async_copy(
                    rowbuf_v.at[0], stage_hbm.at[0], sem).wait()
            pltpu.async_copy(rowbuf_v.at[slot], stage_hbm.at[b], sem)
            return m & (lanes != lane), c + 1

        def cond(carry):
            return has(carry[0])

        _, cnt = lax.while_loop(cond, body, (wmask, cnt))
        return cnt

    return lax.fori_loop(0, nv, jstep, cnt0)


def _drain(rowbuf_v, stage_hbm, sem, cnt):
    def dstep(_, c):
        pltpu.make_async_copy(rowbuf_v.at[0], stage_hbm.at[0], sem).wait()
        return c
    lax.fori_loop(0, jnp.minimum(cnt, RING), dstep, 0)


def _scan_body(uidx_hbm, iidx_hbm, utab_hbm, itab_hbm, ustage_hbm,
               istage_hbm, uidx_v, iidx_v, uhits_v, ihits_v,
               uwin_v, iwin_v, uwin2_v, iwin2_v, uwin64_v, iwin64_v,
               urow_v, irow_v, sp_v, usem, isem, usem2, isem2,
               ulsem, ilsem, ulsem2, ilsem2, ustsem, istsem):
    wid = lax.axis_index("s") * NC + lax.axis_index("c")
    sid = lax.axis_index("s")
    r0 = wid * RSZ
    rsz = jnp.where(wid == 31, V - TAIL0, RSZ)

    pltpu.sync_copy(uidx_hbm, uidx_v)
    pltpu.sync_copy(iidx_hbm, iidx_v)

    n_u = _scan_hits(uidx_v, uhits_v, r0, rsz)
    n_i = _scan_hits(iidx_v, ihits_v, r0, rsz)
    nv_u = (n_u + L - 1) // L
    nv_i = (n_i + L - 1) // L

    hsems = ((usem, isem), (usem2, isem2))
    lsems = ((ulsem, ilsem), (ulsem2, ilsem2))
    tbufs = ((uwin_v, iwin_v), (uwin2_v, iwin2_v))

    def fire_hbm(w, p):
        woff = r0 + w * WIN
        pltpu.async_copy(utab_hbm.at[:, pl.ds(woff, WIN)],
                         sp_v.at[sid, p, 0], hsems[p][0])
        pltpu.async_copy(itab_hbm.at[:, pl.ds(woff, WIN)],
                         sp_v.at[sid, p, 1], hsems[p][1])

    def wait_hbm(p):
        pltpu.make_async_copy(utab_hbm.at[:, pl.ds(0, WIN)],
                              sp_v.at[sid, p, 0], hsems[p][0]).wait()
        pltpu.make_async_copy(itab_hbm.at[:, pl.ds(0, WIN)],
                              sp_v.at[sid, p, 1], hsems[p][1]).wait()

    def fire_loc(p):
        pltpu.async_copy(sp_v.at[sid, p, 0], tbufs[p][0], lsems[p][0])
        pltpu.async_copy(sp_v.at[sid, p, 1], tbufs[p][1], lsems[p][1])

    def wait_loc(p):
        pltpu.make_async_copy(sp_v.at[sid, p, 0], tbufs[p][0],
                              lsems[p][0]).wait()
        pltpu.make_async_copy(sp_v.at[sid, p, 1], tbufs[p][1],
                              lsems[p][1]).wait()

    def extract(w, ubuf, ibuf, cu, ci):
        cu = _extract_window(ubuf, uhits_v, nv_u, w, WIN, urow_v,
                             ustage_hbm, ustsem, cu)
        ci = _extract_window(ibuf, ihits_v, nv_i, w, WIN, irow_v,
                             istage_hbm, istsem, ci)
        return cu, ci

    fire_hbm(0, 0)
    fire_hbm(1, 1)
    wait_hbm(0)
    fire_loc(0)

    def pairstep(t, carry):
        cu, ci = carry
        wait_hbm(1)
        fire_loc(1)
        wait_loc(0)

        @pl.when(t < NWIN // 2 - 1)
        def _():
            fire_hbm(2 * t + 2, 0)
        cu, ci = extract(2 * t, tbufs[0][0], tbufs[0][1], cu, ci)

        @pl.when(t < NWIN // 2 - 1)
        def _():
            wait_hbm(0)
            fire_loc(0)
        wait_loc(1)

        @pl.when(t < NWIN // 2 - 1)
        def _():
            fire_hbm(2 * t + 3, 1)
        cu, ci = extract(2 * t + 1, tbufs[1][0], tbufs[1][1], cu, ci)
        return cu, ci

    cu, ci = lax.fori_loop(0, NWIN // 2, pairstep, (0, 0))

    def window(w, carry):
        cu, ci = carry
        woff = r0 + w * WIN
        cp_u = pltpu.async_copy(
            utab_hbm.at[:, pl.ds(woff, WIN)], uwin_v, usem)
        cp_i = pltpu.async_copy(
            itab_hbm.at[:, pl.ds(woff, WIN)], iwin_v, isem)
        cp_u.wait()
        cp_i.wait()
        return extract(w, uwin_v, iwin_v, cu, ci)

    @pl.when(wid == 31)
    def _tail():
        cu2, ci2 = lax.fori_loop(NWIN, NWIN + 4, window, (cu, ci))
        woff = TAIL0 + (NWIN + 4) * WIN
        cp_u = pltpu.async_copy(
            utab_hbm.at[:, pl.ds(woff, 64)], uwin64_v, usem)
        cp_i = pltpu.async_copy(
            itab_hbm.at[:, pl.ds(woff, 64)], iwin64_v, isem)
        cp_u.wait()
        cp_i.wait()
        cu2 = _extract_window(uwin64_v, uhits_v, nv_u, NWIN + 4, 64, urow_v,
                              ustage_hbm, ustsem, cu2)
        ci2 = _extract_window(iwin64_v, ihits_v, nv_i, NWIN + 4, 64, irow_v,
                              istage_hbm, istsem, ci2)
        _drain(urow_v, ustage_hbm, ustsem, cu2)
        _drain(irow_v, istage_hbm, istsem, ci2)

    @pl.when(wid != 31)
    def _nodrain():
        _drain(urow_v, ustage_hbm, ustsem, cu)
        _drain(irow_v, istage_hbm, istsem, ci)


_scan_kernel = functools.partial(
    pl.kernel,
    out_type=(jax.ShapeDtypeStruct((B, SD), jnp.float32),
              jax.ShapeDtypeStruct((B, SD), jnp.float32)),
    mesh=plsc.VectorSubcoreMesh(core_axis_name="c", subcore_axis_name="s"),
    scratch_types=[
        pltpu.VMEM((B,), jnp.int32),
        pltpu.VMEM((B,), jnp.int32),
        pltpu.VMEM((HITCAP,), jnp.int32),
        pltpu.VMEM((HITCAP,), jnp.int32),
        pltpu.VMEM((D, WIN), jnp.float32),
        pltpu.VMEM((D, WIN), jnp.float32),
        pltpu.VMEM((D, WIN), jnp.float32),
        pltpu.VMEM((D, WIN), jnp.float32),
        pltpu.VMEM((D, 64), jnp.float32),
        pltpu.VMEM((D, 64), jnp.float32),
        pltpu.VMEM((RING, SD), jnp.float32),
        pltpu.VMEM((RING, SD), jnp.float32),
        pltpu.VMEM_SHARED((NS, 2, 2, D, WIN), jnp.float32),
        pltpu.SemaphoreType.DMA,
        pltpu.SemaphoreType.DMA,
        pltpu.SemaphoreType.DMA,
        pltpu.SemaphoreType.DMA,
        pltpu.SemaphoreType.DMA,
        pltpu.SemaphoreType.DMA,
        pltpu.SemaphoreType.DMA,
        pltpu.SemaphoreType.DMA,
        pltpu.SemaphoreType.DMA,
        pltpu.SemaphoreType.DMA,
    ],
    compiler_params=pltpu.CompilerParams(
        needs_layout_passes=False, use_tc_tiling_on_sc=True),
)(_scan_body)


CH = 128
BPW = B // NW


def _dot_body(ustage_hbm, istage_hbm, out_hbm, urows_v, irows_v,
              urows2_v, irows2_v, out_v, usem, isem, usem2, isem2):
    wid = lax.axis_index("s") * NC + lax.axis_index("c")
    base = wid * BPW
    lanes = lax.iota(jnp.int32, L)

    def fire(k, ubuf, ibuf, us, isem_):
        pltpu.async_copy(ustage_hbm.at[pl.ds(base + k * CH, CH)], ubuf, us)
        pltpu.async_copy(istage_hbm.at[pl.ds(base + k * CH, CH)], ibuf,
                         isem_)

    def wait(ubuf, ibuf, us, isem_):
        pltpu.make_async_copy(ustage_hbm.at[pl.ds(0, CH)], ubuf, us).wait()
        pltpu.make_async_copy(istage_hbm.at[pl.ds(0, CH)], ibuf,
                              isem_).wait()

    def compute(k, ubuf, ibuf):
        def group(j, _):
            ridx = j * L + lanes

            def dstep(d4, acc):
                for q in range(4):
                    cidx = jnp.full((L,), 0, jnp.int32) + (d4 * 4 + q)
                    u = plsc.load_gather(ubuf, [ridx, cidx])
                    v = plsc.load_gather(ibuf, [ridx, cidx])
                    acc = acc + u * v
                return acc

            acc = lax.fori_loop(0, D // 4, dstep,
                                jnp.zeros((L,), jnp.float32))
            out_v[pl.ds(k * CH + j * L, L)] = acc
            return 0
        lax.fori_loop(0, CH // L, group, 0)

    fire(0, urows_v, irows_v, usem, isem)

    def pairstep(t, _):
        fire(2 * t + 1, urows2_v, irows2_v, usem2, isem2)
        wait(urows_v, irows_v, usem, isem)
        compute(2 * t, urows_v, irows_v)

        @pl.when(t < BPW // CH // 2 - 1)
        def _():
            fire(2 * t + 2, urows_v, irows_v, usem, isem)
        wait(urows2_v, irows2_v, usem2, isem2)
        compute(2 * t + 1, urows2_v, irows2_v)
        return 0

    lax.fori_loop(0, BPW // CH // 2, pairstep, 0)
    pltpu.sync_copy(out_v, out_hbm.at[pl.ds(base, BPW)])


_dot_kernel = functools.partial(
    pl.kernel,
    out_type=jax.ShapeDtypeStruct((B,), jnp.float32),
    mesh=plsc.VectorSubcoreMesh(core_axis_name="c", subcore_axis_name="s"),
    scratch_types=[
        pltpu.VMEM((CH, SD), jnp.float32),
        pltpu.VMEM((CH, SD), jnp.float32),
        pltpu.VMEM((CH, SD), jnp.float32),
        pltpu.VMEM((CH, SD), jnp.float32),
        pltpu.VMEM((BPW,), jnp.float32),
        pltpu.SemaphoreType.DMA,
        pltpu.SemaphoreType.DMA,
        pltpu.SemaphoreType.DMA,
        pltpu.SemaphoreType.DMA,
    ],
    compiler_params=pltpu.CompilerParams(
        needs_layout_passes=False, use_tc_tiling_on_sc=True),
)(_dot_body)


def kernel(user_idx, item_idx, user_table, item_table):
    ustage, istage = _scan_kernel(
        user_idx.astype(jnp.int32), item_idx.astype(jnp.int32),
        user_table.T, item_table.T)
    return _dot_kernel(ustage, istage)


# speedup vs baseline: 1.5418x; 1.5418x over previous
import functools

import jax
import jax.numpy as jnp
from jax import lax
from jax.experimental import pallas as pl
from jax.experimental.pallas import tpu as pltpu
from jax.experimental.pallas import tpu_sc as plsc

B = 16384
D = 64
L = 16
NC = 2
NS = 16
NW = NC * NS
V = 1000000
SD = 64

PANELS = 61
WIN = 256
RNG = PANELS * WIN
NWIN = 122
RSZ = NWIN * WIN
TAIL0 = 31 * RSZ

HITCAP = 1024
RING = 8

_SENTINEL = 0x3FFFFFFF


def _scan_hits(idx_v, hits_v, r0, rsz):
    big = jnp.full((L,), _SENTINEL, jnp.int32)

    def prefill(k, _):
        hits_v[pl.ds(k * L, L)] = big
        return 0
    lax.fori_loop(0, HITCAP // L, prefill, 0)

    lanes = lax.iota(jnp.int32, L)

    def step(j, n):
        vec = idx_v[pl.ds(j * L, L)]
        rl = vec - r0
        mask = (rl >= 0) & (rl < rsz)
        key = rl * 16384 + (j * L + lanes)
        plsc.store_compressed(hits_v.at[pl.ds(n, L)], key, mask=mask)
        cnt = plsc.all_reduce_population_count(mask)[0]
        return jnp.minimum(n + cnt, HITCAP - L)

    return lax.fori_loop(0, B // L, step, 0)


def _extract_window(win_v, hits_v, nv, w, width, rowbuf_v, stage_hbm,
                    sem, cnt0):
    lanes = lax.iota(jnp.int32, L)

    def jstep(jv, cnt):
        kvec = hits_v[pl.ds(jv * L, L)]
        wmask = (kvec >> 22) == w

        def has(m):
            return plsc.all_reduce_population_count(m)[0] > 0

        def body(carry):
            m, c = carry
            lane = plsc.all_reduce_ffs(m)[0]
            keyv = jnp.take_along_axis(
                kvec, jnp.full((L,), 0, jnp.int32) + lane, axis=0,
                mode="promise_in_bounds")
            relv = (keyv >> 14) - w * WIN
            b = keyv[0] & 16383
            slot = c & (RING - 1)
            for g in range(D // L):
                vals = plsc.load_gather(win_v, [g * L + lanes, relv])
                rowbuf_v[slot, pl.ds(g * L, L)] = vals
            @pl.when(c >= RING)
            def _():
                pltpu.make_async_copy(
                    rowbuf_v.at[0], stage_hbm.at[0], sem).wait()
            pltpu.async_copy(rowbuf_v.at[slot], stage_hbm.at[b], sem)
            return m & (lanes != lane), c + 1

        def cond(carry):
            return has(carry[0])

        _, cnt = lax.while_loop(cond, body, (wmask, cnt))
        return cnt

    return lax.fori_loop(0, nv, jstep, cnt0)


def _drain(rowbuf_v, stage_hbm, sem, cnt):
    def dstep(_, c):
        pltpu.make_async_copy(rowbuf_v.at[0], stage_hbm.at[0], sem).wait()
        return c
    lax.fori_loop(0, jnp.minimum(cnt, RING), dstep, 0)


def _scan_body(uidx_hbm, iidx_hbm, utab_hbm, itab_hbm, ustage_hbm,
               istage_hbm, uidx_v, iidx_v, uhits_v, ihits_v,
               uwin_v, iwin_v, uwin2_v, iwin2_v, uwin64_v, iwin64_v,
               urow_v, irow_v, usem, isem, usem2, isem2, ustsem, istsem):
    wid = lax.axis_index("s") * NC + lax.axis_index("c")
    r0 = wid * RSZ
    rsz = jnp.where(wid == 31, V - TAIL0, RSZ)

    pltpu.sync_copy(uidx_hbm, uidx_v)
    pltpu.sync_copy(iidx_hbm, iidx_v)

    n_u = _scan_hits(uidx_v, uhits_v, r0, rsz)
    n_i = _scan_hits(iidx_v, ihits_v, r0, rsz)
    nv_u = (n_u + L - 1) // L
    nv_i = (n_i + L - 1) // L

    def fire(w, ubuf, ibuf, us, isem_):
        woff = r0 + w * WIN
        pltpu.async_copy(utab_hbm.at[:, pl.ds(woff, WIN)], ubuf, us)
        pltpu.async_copy(itab_hbm.at[:, pl.ds(woff, WIN)], ibuf, isem_)

    def wait(ubuf, ibuf, us, isem_):
        pltpu.make_async_copy(utab_hbm.at[:, pl.ds(0, WIN)], ubuf, us).wait()
        pltpu.make_async_copy(itab_hbm.at[:, pl.ds(0, WIN)], ibuf,
                              isem_).wait()

    def extract(w, ubuf, ibuf, cu, ci):
        cu = _extract_window(ubuf, uhits_v, nv_u, w, WIN, urow_v,
                             ustage_hbm, ustsem, cu)
        ci = _extract_window(ibuf, ihits_v, nv_i, w, WIN, irow_v,
                             istage_hbm, istsem, ci)
        return cu, ci

    fire(0, uwin_v, iwin_v, usem, isem)

    def pairstep(t, carry):
        cu, ci = carry
        fire(2 * t + 1, uwin2_v, iwin2_v, usem2, isem2)
        wait(uwin_v, iwin_v, usem, isem)
        cu, ci = extract(2 * t, uwin_v, iwin_v, cu, ci)

        @pl.when(t < NWIN // 2 - 1)
        def _():
            fire(2 * t + 2, uwin_v, iwin_v, usem, isem)
        wait(uwin2_v, iwin2_v, usem2, isem2)
        cu, ci = extract(2 * t + 1, uwin2_v, iwin2_v, cu, ci)
        return cu, ci

    cu, ci = lax.fori_loop(0, NWIN // 2, pairstep, (0, 0))

    def window(w, carry):
        cu, ci = carry
        fire(w, uwin_v, iwin_v, usem, isem)
        wait(uwin_v, iwin_v, usem, isem)
        return extract(w, uwin_v, iwin_v, cu, ci)

    @pl.when(wid == 31)
    def _tail():
        cu2, ci2 = lax.fori_loop(NWIN, NWIN + 2, window, (cu, ci))
        woff = TAIL0 + 124 * WIN
        cp_u = pltpu.async_copy(
            utab_hbm.at[:, pl.ds(woff, 64)], uwin64_v, usem)
        cp_i = pltpu.async_copy(
            itab_hbm.at[:, pl.ds(woff, 64)], iwin64_v, isem)
        cp_u.wait()
        cp_i.wait()
        cu2 = _extract_window(uwin64_v, uhits_v, nv_u, 124, 64, urow_v,
                              ustage_hbm, ustsem, cu2)
        ci2 = _extract_window(iwin64_v, ihits_v, nv_i, 124, 64, irow_v,
                              istage_hbm, istsem, ci2)
        _drain(urow_v, ustage_hbm, ustsem, cu2)
        _drain(irow_v, istage_hbm, istsem, ci2)

    @pl.when(wid != 31)
    def _nodrain():
        _drain(urow_v, ustage_hbm, ustsem, cu)
        _drain(irow_v, istage_hbm, istsem, ci)


_scan_kernel = functools.partial(
    pl.kernel,
    out_type=(jax.ShapeDtypeStruct((B, SD), jnp.float32),
              jax.ShapeDtypeStruct((B, SD), jnp.float32)),
    mesh=plsc.VectorSubcoreMesh(core_axis_name="c", subcore_axis_name="s"),
    scratch_types=[
        pltpu.VMEM((B,), jnp.int32),
        pltpu.VMEM((B,), jnp.int32),
        pltpu.VMEM((HITCAP,), jnp.int32),
        pltpu.VMEM((HITCAP,), jnp.int32),
        pltpu.VMEM((D, WIN), jnp.float32),
        pltpu.VMEM((D, WIN), jnp.float32),
        pltpu.VMEM((D, WIN), jnp.float32),
        pltpu.VMEM((D, WIN), jnp.float32),
        pltpu.VMEM((D, 64), jnp.float32),
        pltpu.VMEM((D, 64), jnp.float32),
        pltpu.VMEM((RING, SD), jnp.float32),
        pltpu.VMEM((RING, SD), jnp.float32),
        pltpu.SemaphoreType.DMA,
        pltpu.SemaphoreType.DMA,
        pltpu.SemaphoreType.DMA,
        pltpu.SemaphoreType.DMA,
        pltpu.SemaphoreType.DMA,
        pltpu.SemaphoreType.DMA,
    ],
    compiler_params=pltpu.CompilerParams(
        needs_layout_passes=False, use_tc_tiling_on_sc=True),
)(_scan_body)


CH = 128
BPW = B // NW


def _dot_body(ustage_hbm, istage_hbm, out_hbm, urows_v, irows_v,
              urows2_v, irows2_v, out_v, usem, isem, usem2, isem2):
    wid = lax.axis_index("s") * NC + lax.axis_index("c")
    base = wid * BPW
    lanes = lax.iota(jnp.int32, L)

    def fire(k, ubuf, ibuf, us, isem_):
        pltpu.async_copy(ustage_hbm.at[pl.ds(base + k * CH, CH)], ubuf, us)
        pltpu.async_copy(istage_hbm.at[pl.ds(base + k * CH, CH)], ibuf,
                         isem_)

    def wait(ubuf, ibuf, us, isem_):
        pltpu.make_async_copy(ustage_hbm.at[pl.ds(0, CH)], ubuf, us).wait()
        pltpu.make_async_copy(istage_hbm.at[pl.ds(0, CH)], ibuf,
                              isem_).wait()

    def compute(k, ubuf, ibuf):
        def group(j, _):
            ridx = j * L + lanes

            def dstep(d4, acc):
                for q in range(4):
                    cidx = jnp.full((L,), 0, jnp.int32) + (d4 * 4 + q)
                    u = plsc.load_gather(ubuf, [ridx, cidx])
                    v = plsc.load_gather(ibuf, [ridx, cidx])
                    acc = acc + u * v
                return acc

            acc = lax.fori_loop(0, D // 4, dstep,
                                jnp.zeros((L,), jnp.float32))
            out_v[pl.ds(k * CH + j * L, L)] = acc
            return 0
        lax.fori_loop(0, CH // L, group, 0)

    fire(0, urows_v, irows_v, usem, isem)

    def pairstep(t, _):
        fire(2 * t + 1, urows2_v, irows2_v, usem2, isem2)
        wait(urows_v, irows_v, usem, isem)
        compute(2 * t, urows_v, irows_v)

        @pl.when(t < BPW // CH // 2 - 1)
        def _():
            fire(2 * t + 2, urows_v, irows_v, usem, isem)
        wait(urows2_v, irows2_v, usem2, isem2)
        compute(2 * t + 1, urows2_v, irows2_v)
        return 0

    lax.fori_loop(0, BPW // CH // 2, pairstep, 0)
    pltpu.sync_copy(out_v, out_hbm.at[pl.ds(base, BPW)])


_dot_kernel = functools.partial(
    pl.kernel,
    out_type=jax.ShapeDtypeStruct((B,), jnp.float32),
    mesh=plsc.VectorSubcoreMesh(core_axis_name="c", subcore_axis_name="s"),
    scratch_types=[
        pltpu.VMEM((CH, SD), jnp.float32),
        pltpu.VMEM((CH, SD), jnp.float32),
        pltpu.VMEM((CH, SD), jnp.float32),
        pltpu.VMEM((CH, SD), jnp.float32),
        pltpu.VMEM((BPW,), jnp.float32),
        pltpu.SemaphoreType.DMA,
        pltpu.SemaphoreType.DMA,
        pltpu.SemaphoreType.DMA,
        pltpu.SemaphoreType.DMA,
    ],
    compiler_params=pltpu.CompilerParams(
        needs_layout_passes=False, use_tc_tiling_on_sc=True),
)(_dot_body)


def kernel(user_idx, item_idx, user_table, item_table):
    ustage, istage = _scan_kernel(
        user_idx.astype(jnp.int32), item_idx.astype(jnp.int32),
        user_table.T, item_table.T)
    return _dot_kernel(ustage, istage)


# speedup vs baseline: 1.6607x; 1.0771x over previous
import functools

import jax
import jax.numpy as jnp
from jax import lax
from jax.experimental import pallas as pl
from jax.experimental.pallas import tpu as pltpu
from jax.experimental.pallas import tpu_sc as plsc

B = 16384
D = 64
L = 16
NC = 2
NS = 16
NW = NC * NS
V = 1000000
SD = 64

PANELS = 61
WIN = 256
RNG = PANELS * WIN
NWIN = 122
RSZ = NWIN * WIN
TAIL0 = 31 * RSZ

HITCAP = 1024
RING = 8

_SENTINEL = 0x3FFFFFFF


def _scan_hits(idx_v, hits_v, r0, rsz):
    big = jnp.full((L,), _SENTINEL, jnp.int32)

    def prefill(k, _):
        hits_v[pl.ds(k * L, L)] = big
        return 0
    lax.fori_loop(0, HITCAP // L, prefill, 0)

    lanes = lax.iota(jnp.int32, L)

    def step(j, n):
        vec = idx_v[pl.ds(j * L, L)]
        rl = vec - r0
        mask = (rl >= 0) & (rl < rsz)
        key = rl * 16384 + (j * L + lanes)
        plsc.store_compressed(hits_v.at[pl.ds(n, L)], key, mask=mask)
        cnt = plsc.all_reduce_population_count(mask)[0]
        return jnp.minimum(n + cnt, HITCAP - L)

    return lax.fori_loop(0, B // L, step, 0)


def _extract_window(win_v, hits_v, nv, w, width, rowbuf_v, stage_hbm,
                    sem, cnt0):
    lanes = lax.iota(jnp.int32, L)

    def jstep(jv, cnt):
        kvec = hits_v[pl.ds(jv * L, L)]
        wmask = (kvec >> 22) == w

        def has(m):
            return plsc.all_reduce_population_count(m)[0] > 0

        def body(carry):
            m, c = carry
            lane = plsc.all_reduce_ffs(m)[0]
            keyv = jnp.take_along_axis(
                kvec, jnp.full((L,), 0, jnp.int32) + lane, axis=0,
                mode="promise_in_bounds")
            relv = (keyv >> 14) - w * WIN
            b = keyv[0] & 16383
            slot = c & (RING - 1)
            for g in range(D // L):
                vals = plsc.load_gather(win_v, [g * L + lanes, relv])
                rowbuf_v[slot, pl.ds(g * L, L)] = vals
            @pl.when(c >= RING)
            def _():
                pltpu.make_async_copy(
                    rowbuf_v.at[0], stage_hbm.at[0], sem).wait()
            pltpu.async_copy(rowbuf_v.at[slot], stage_hbm.at[b], sem)
            return m & (lanes != lane), c + 1

        def cond(carry):
            return has(carry[0])

        _, cnt = lax.while_loop(cond, body, (wmask, cnt))
        return cnt

    return lax.fori_loop(0, nv, jstep, cnt0)


def _drain(rowbuf_v, stage_hbm, sem, cnt):
    def dstep(_, c):
        pltpu.make_async_copy(rowbuf_v.at[0], stage_hbm.at[0], sem).wait()
        return c
    lax.fori_loop(0, jnp.minimum(cnt, RING), dstep, 0)


def _scan_body(uidx_hbm, iidx_hbm, utab_hbm, itab_hbm, ustage_hbm,
               istage_hbm, uidx_v, iidx_v, uhits_v, ihits_v,
               uwin_v, iwin_v, uwin2_v, iwin2_v, uwin64_v, iwin64_v,
               urow_v, irow_v, usem, isem, usem2, isem2, ustsem, istsem):
    wid = lax.axis_index("s") * NC + lax.axis_index("c")
    r0 = wid * RSZ
    rsz = jnp.where(wid == 31, V - TAIL0, RSZ)

    pltpu.sync_copy(uidx_hbm, uidx_v)
    pltpu.sync_copy(iidx_hbm, iidx_v)

    n_u = _scan_hits(uidx_v, uhits_v, r0, rsz)
    n_i = _scan_hits(iidx_v, ihits_v, r0, rsz)
    nv_u = (n_u + L - 1) // L
    nv_i = (n_i + L - 1) // L

    def fire(w, ubuf, ibuf, us, isem_):
        woff = r0 + w * WIN
        pltpu.async_copy(utab_hbm.at[:, pl.ds(woff, WIN)], ubuf, us)
        pltpu.async_copy(itab_hbm.at[:, pl.ds(woff, WIN)], ibuf, isem_)

    def wait(ubuf, ibuf, us, isem_):
        pltpu.make_async_copy(utab_hbm.at[:, pl.ds(0, WIN)], ubuf, us).wait()
        pltpu.make_async_copy(itab_hbm.at[:, pl.ds(0, WIN)], ibuf,
                              isem_).wait()

    def extract(w, ubuf, ibuf, cu, ci):
        cu = _extract_window(ubuf, uhits_v, nv_u, w, WIN, urow_v,
                             ustage_hbm, ustsem, cu)
        ci = _extract_window(ibuf, ihits_v, nv_i, w, WIN, irow_v,
                             istage_hbm, istsem, ci)
        return cu, ci

    fire(0, uwin_v, iwin_v, usem, isem)

    def pairstep(t, carry):
        cu, ci = carry
        fire(2 * t + 1, uwin2_v, iwin2_v, usem2, isem2)
        wait(uwin_v, iwin_v, usem, isem)
        cu, ci = extract(2 * t, uwin_v, iwin_v, cu, ci)

        @pl.when(t < NWIN // 2 - 1)
        def _():
            fire(2 * t + 2, uwin_v, iwin_v, usem, isem)
        wait(uwin2_v, iwin2_v, usem2, isem2)
        cu, ci = extract(2 * t + 1, uwin2_v, iwin2_v, cu, ci)
        return cu, ci

    cu, ci = lax.fori_loop(0, NWIN // 2, pairstep, (0, 0))

    def window(w, carry):
        cu, ci = carry
        fire(w, uwin_v, iwin_v, usem, isem)
        wait(uwin_v, iwin_v, usem, isem)
        return extract(w, uwin_v, iwin_v, cu, ci)

    @pl.when(wid == 31)
    def _tail():
        cu2, ci2 = lax.fori_loop(NWIN, NWIN + 2, window, (cu, ci))
        woff = TAIL0 + 124 * WIN
        cp_u = pltpu.async_copy(
            utab_hbm.at[:, pl.ds(woff, 64)], uwin64_v, usem)
        cp_i = pltpu.async_copy(
            itab_hbm.at[:, pl.ds(woff, 64)], iwin64_v, isem)
        cp_u.wait()
        cp_i.wait()
        cu2 = _extract_window(uwin64_v, uhits_v, nv_u, 124, 64, urow_v,
                              ustage_hbm, ustsem, cu2)
        ci2 = _extract_window(iwin64_v, ihits_v, nv_i, 124, 64, irow_v,
                              istage_hbm, istsem, ci2)
        _drain(urow_v, ustage_hbm, ustsem, cu2)
        _drain(irow_v, istage_hbm, istsem, ci2)

    @pl.when(wid != 31)
    def _nodrain():
        _drain(urow_v, ustage_hbm, ustsem, cu)
        _drain(irow_v, istage_hbm, istsem, ci)


_scan_kernel = functools.partial(
    pl.kernel,
    out_type=(jax.ShapeDtypeStruct((B, SD), jnp.float32),
              jax.ShapeDtypeStruct((B, SD), jnp.float32)),
    mesh=plsc.VectorSubcoreMesh(core_axis_name="c", subcore_axis_name="s"),
    scratch_types=[
        pltpu.VMEM((B,), jnp.int32),
        pltpu.VMEM((B,), jnp.int32),
        pltpu.VMEM((HITCAP,), jnp.int32),
        pltpu.VMEM((HITCAP,), jnp.int32),
        pltpu.VMEM((D, WIN), jnp.float32),
        pltpu.VMEM((D, WIN), jnp.float32),
        pltpu.VMEM((D, WIN), jnp.float32),
        pltpu.VMEM((D, WIN), jnp.float32),
        pltpu.VMEM((D, 64), jnp.float32),
        pltpu.VMEM((D, 64), jnp.float32),
        pltpu.VMEM((RING, SD), jnp.float32),
        pltpu.VMEM((RING, SD), jnp.float32),
        pltpu.SemaphoreType.DMA,
        pltpu.SemaphoreType.DMA,
        pltpu.SemaphoreType.DMA,
        pltpu.SemaphoreType.DMA,
        pltpu.SemaphoreType.DMA,
        pltpu.SemaphoreType.DMA,
    ],
    compiler_params=pltpu.CompilerParams(
        needs_layout_passes=False, use_tc_tiling_on_sc=True),
)(_scan_body)


CH = 128
BPW = B // NW


def _dot_body(ustage_hbm, istage_hbm, out_hbm, urows_v, irows_v,
              urows2_v, irows2_v, out_v, usem, isem, usem2, isem2):
    wid = lax.axis_index("s") * NC + lax.axis_index("c")
    base = wid * BPW
    lanes = lax.iota(jnp.int32, L)

    def fire(k, ubuf, ibuf, us, isem_):
        pltpu.async_copy(ustage_hbm.at[pl.ds(base + k * CH, CH)], ubuf, us)
        pltpu.async_copy(istage_hbm.at[pl.ds(base + k * CH, CH)], ibuf,
                         isem_)

    def wait(ubuf, ibuf, us, isem_):
        pltpu.make_async_copy(ustage_hbm.at[pl.ds(0, CH)], ubuf, us).wait()
        pltpu.make_async_copy(istage_hbm.at[pl.ds(0, CH)], ibuf,
                              isem_).wait()

    def compute(k, ubuf, ibuf):
        def group(j, _):
            ridx = j * L + lanes

            def dstep(d4, acc):
                for q in range(4):
                    cidx = jnp.full((L,), 0, jnp.int32) + (d4 * 4 + q)
                    u = plsc.load_gather(ubuf, [ridx, cidx])
                    v = plsc.load_gather(ibuf, [ridx, cidx])
                    acc = acc + u * v
                return acc

            acc = lax.fori_loop(0, D // 4, dstep,
                                jnp.zeros((L,), jnp.float32))
            out_v[pl.ds(k * CH + j * L, L)] = acc
            return 0
        lax.fori_loop(0, CH // L, group, 0)

    fire(0, urows_v, irows_v, usem, isem)

    def pairstep(t, _):
        fire(2 * t + 1, urows2_v, irows2_v, usem2, isem2)
        wait(urows_v, irows_v, usem, isem)
        compute(2 * t, urows_v, irows_v)

        @pl.when(t < BPW // CH // 2 - 1)
        def _():
            fire(2 * t + 2, urows_v, irows_v, usem, isem)
        wait(urows2_v, irows2_v, usem2, isem2)
        compute(2 * t + 1, urows2_v, irows2_v)
        return 0

    lax.fori_loop(0, BPW // CH // 2, pairstep, 0)
    pltpu.sync_copy(out_v, out_hbm.at[pl.ds(base, BPW)])


_dot_kernel = functools.partial(
    pl.kernel,
    out_type=jax.ShapeDtypeStruct((B,), jnp.float32),
    mesh=plsc.VectorSubcoreMesh(core_axis_name="c", subcore_axis_name="s"),
    scratch_types=[
        pltpu.VMEM((CH, SD), jnp.float32),
        pltpu.VMEM((CH, SD), jnp.float32),
        pltpu.VMEM((CH, SD), jnp.float32),
        pltpu.VMEM((CH, SD), jnp.float32),
        pltpu.VMEM((BPW,), jnp.float32),
        pltpu.SemaphoreType.DMA,
        pltpu.SemaphoreType.DMA,
        pltpu.SemaphoreType.DMA,
        pltpu.SemaphoreType.DMA,
    ],
    compiler_params=pltpu.CompilerParams(
        needs_layout_passes=False, use_tc_tiling_on_sc=True),
)(_dot_body)


_TCB = 2048


def _dot_tc_body(u_ref, i_ref, o_ref):
    o_ref[...] = jnp.sum(u_ref[...] * i_ref[...], axis=1)


_dot_tc = pl.pallas_call(
    _dot_tc_body,
    grid=(B // _TCB,),
    in_specs=[
        pl.BlockSpec((_TCB, SD), lambda i: (i, 0)),
        pl.BlockSpec((_TCB, SD), lambda i: (i, 0)),
    ],
    out_specs=pl.BlockSpec((_TCB,), lambda i: (i,)),
    out_shape=jax.ShapeDtypeStruct((B,), jnp.float32),
)


def kernel(user_idx, item_idx, user_table, item_table):
    ustage, istage = _scan_kernel(
        user_idx.astype(jnp.int32), item_idx.astype(jnp.int32),
        user_table.T, item_table.T)
    return _dot_tc(ustage, istage)
